# Initial kernel scaffold; baseline (speedup 1.0000x reference)
import jax
import jax.numpy as jnp
from jax.experimental import pallas as pl


def kernel(x, edge_index, edge_attr, W1, b1, W2, b2):
    raise NotImplementedError("write your pallas kernel here")


# speedup vs baseline: 6.8120x; 6.8120x over previous
import functools

import jax
import jax.numpy as jnp
from jax import lax
from jax.experimental import pallas as pl
from jax.experimental.pallas import tpu as pltpu
from jax.experimental.pallas import tpu_sc as plsc

N = 10000
D = 128
E = 320000
NC, NS, L = 2, 16, 16
NW = NC * NS
K = 128
CH_T = 79
EPT = CH_T * K
EP = EPT * NW
NPAD = 10240
RPS = NPAD // NS
CSZ = 128

_mesh = plsc.VectorSubcoreMesh(core_axis_name="c", subcore_axis_name="s")


def _deg_body(dst_hbm, attr_hbm, out_hbm, dst_v, attr_v, rows_v, stage_v, dacc):
    c = lax.axis_index("c")
    s = lax.axis_index("s")
    wid = s * NC + c
    zero = jnp.zeros((L,), jnp.float32)

    def zrow(i, _):
        r = i // 8
        col = (i % 8) * L
        stage_v[r, pl.ds(col, L)] = zero
        return 0

    lax.fori_loop(0, CSZ * 8, zrow, 0)

    def zchunk(i, _):
        pltpu.sync_copy(stage_v, dacc.at[pl.ds(s * RPS + i * CSZ, CSZ)])
        return 0

    lax.fori_loop(0, RPS // CSZ, zchunk, 0)
    plsc.subcore_barrier()

    base0 = wid * EPT

    def chunk(i, _):
        base = base0 + i * K
        pltpu.sync_copy(dst_hbm.at[pl.ds(base, K)], dst_v)
        pltpu.sync_copy(attr_hbm.at[pl.ds(base, K)], attr_v)

        def fill(g, _):
            av = attr_v[pl.ds(g * L, L)]
            for lane in range(L):
                e = g * L + lane
                splat = jnp.full((L,), av[lane], jnp.float32)
                for j in range(D // L):
                    rows_v[e, pl.ds(j * L, L)] = splat
            return 0

        lax.fori_loop(0, K // L, fill, 0)
        pltpu.sync_copy(rows_v, dacc.at[dst_v], add=True)
        return 0

    lax.fori_loop(0, CH_T, chunk, 0)
    plsc.subcore_barrier()

    def ochunk(i, _):
        r0 = s * RPS + i * CSZ
        pltpu.sync_copy(dacc.at[pl.ds(r0, CSZ)], stage_v)
        pltpu.sync_copy(stage_v, out_hbm.at[c, pl.ds(r0, CSZ)])
        return 0

    lax.fori_loop(0, RPS // CSZ, ochunk, 0)


_deg_call = pl.kernel(
    _deg_body,
    out_type=jax.ShapeDtypeStruct((NC, NPAD, D), jnp.float32),
    mesh=_mesh,
    scratch_types=[
        pltpu.VMEM((K,), jnp.int32),
        pltpu.VMEM((K,), jnp.float32),
        pltpu.VMEM((K, D), jnp.float32),
        pltpu.VMEM((CSZ, D), jnp.float32),
        pltpu.VMEM_SHARED((NPAD, D), jnp.float32),
    ],
)


def _agg_body(g_hbm, src_hbm, dst_hbm, attr_hbm, out_hbm,
              src_v, dst_v, attr_v, rows_v, stage_v, acc, sem):
    c = lax.axis_index("c")
    s = lax.axis_index("s")
    wid = s * NC + c
    zero = jnp.zeros((L,), jnp.float32)

    def zrow(i, _):
        r = i // 8
        col = (i % 8) * L
        stage_v[r, pl.ds(col, L)] = zero
        return 0

    lax.fori_loop(0, CSZ * 8, zrow, 0)

    def zchunk(i, _):
        pltpu.sync_copy(stage_v, acc.at[pl.ds(s * RPS + i * CSZ, CSZ)])
        return 0

    lax.fori_loop(0, RPS // CSZ, zchunk, 0)
    plsc.subcore_barrier()

    base0 = wid * EPT

    def chunk(i, _):
        base = base0 + i * K
        pltpu.sync_copy(src_hbm.at[pl.ds(base, K)], src_v)
        pltpu.sync_copy(dst_hbm.at[pl.ds(base, K)], dst_v)
        pltpu.sync_copy(attr_hbm.at[pl.ds(base, K)], attr_v)
        pltpu.async_copy(g_hbm.at[src_v], rows_v, sem).wait()

        def scale(g, _):
            av = attr_v[pl.ds(g * L, L)]
            for lane in range(L):
                e = g * L + lane
                a = av[lane]
                for j in range(D // L):
                    sl = pl.ds(j * L, L)
                    rows_v[e, sl] = rows_v[e, sl] * a
            return 0

        lax.fori_loop(0, K // L, scale, 0)
        pltpu.sync_copy(rows_v, acc.at[dst_v], add=True)
        return 0

    lax.fori_loop(0, CH_T, chunk, 0)
    plsc.subcore_barrier()

    def ochunk(i, _):
        r0 = s * RPS + i * CSZ
        pltpu.sync_copy(acc.at[pl.ds(r0, CSZ)], stage_v)
        pltpu.sync_copy(stage_v, out_hbm.at[c, pl.ds(r0, CSZ)])
        return 0

    lax.fori_loop(0, RPS // CSZ, ochunk, 0)


_agg_call = pl.kernel(
    _agg_body,
    out_type=jax.ShapeDtypeStruct((NC, NPAD, D), jnp.float32),
    mesh=_mesh,
    scratch_types=[
        pltpu.VMEM((K,), jnp.int32),
        pltpu.VMEM((K,), jnp.int32),
        pltpu.VMEM((K,), jnp.float32),
        pltpu.VMEM((K, D), jnp.float32),
        pltpu.VMEM((CSZ, D), jnp.float32),
        pltpu.VMEM_SHARED((NPAD, D), jnp.float32),
        pltpu.SemaphoreType.DMA,
    ],
)


def _dis_body(d_ref, o_ref):
    dsum = d_ref[0] + d_ref[1]
    o_ref[...] = jnp.where(dsum > 0, lax.rsqrt(jnp.maximum(dsum, 1e-12)), 0.0)


def _dis_call(degs):
    return pl.pallas_call(
        _dis_body,
        grid=(N // _RB,),
        in_specs=[pl.BlockSpec((NC, _RB, D), lambda i: (0, i, 0))],
        out_specs=pl.BlockSpec((_RB, D), lambda i: (i, 0)),
        out_shape=jax.ShapeDtypeStruct((N, D), jnp.float32),
    )(degs)


_RB = 2000


def _mm1_body(x_ref, w_ref, dis_ref, o_ref):
    o_ref[...] = jnp.dot(x_ref[...], w_ref[...],
                         preferred_element_type=jnp.float32) * dis_ref[...]


def _mm1(x, W, dis):
    return pl.pallas_call(
        _mm1_body,
        grid=(N // _RB,),
        in_specs=[
            pl.BlockSpec((_RB, D), lambda i: (i, 0)),
            pl.BlockSpec((D, D), lambda i: (0, 0)),
            pl.BlockSpec((_RB, D), lambda i: (i, 0)),
        ],
        out_specs=pl.BlockSpec((_RB, D), lambda i: (i, 0)),
        out_shape=jax.ShapeDtypeStruct((N, D), jnp.float32),
    )(x, W, dis)


def _mm2_body(p_ref, dis_ref, b_ref, w_ref, o_ref):
    a = (p_ref[0] + p_ref[1]) * dis_ref[...] + b_ref[...]
    o_ref[...] = jnp.dot(a, w_ref[...],
                         preferred_element_type=jnp.float32) * dis_ref[...]


def _mm2(p, dis, b, W):
    return pl.pallas_call(
        _mm2_body,
        grid=(N // _RB,),
        in_specs=[
            pl.BlockSpec((NC, _RB, D), lambda i: (0, i, 0)),
            pl.BlockSpec((_RB, D), lambda i: (i, 0)),
            pl.BlockSpec((1, D), lambda i: (0, 0)),
            pl.BlockSpec((D, D), lambda i: (0, 0)),
        ],
        out_specs=pl.BlockSpec((_RB, D), lambda i: (i, 0)),
        out_shape=jax.ShapeDtypeStruct((N, D), jnp.float32),
    )(p, dis, b, W)


def _fin_body(q_ref, dis_ref, b_ref, o_ref):
    o_ref[...] = (q_ref[0] + q_ref[1]) * dis_ref[...] + b_ref[...]


def _fin(q, dis, b):
    return pl.pallas_call(
        _fin_body,
        grid=(N // _RB,),
        in_specs=[
            pl.BlockSpec((NC, _RB, D), lambda i: (0, i, 0)),
            pl.BlockSpec((_RB, D), lambda i: (i, 0)),
            pl.BlockSpec((1, D), lambda i: (0, 0)),
        ],
        out_specs=pl.BlockSpec((_RB, D), lambda i: (i, 0)),
        out_shape=jax.ShapeDtypeStruct((N, D), jnp.float32),
    )(q, dis, b)


def kernel(x, edge_index, edge_attr, W1, b1, W2, b2):
    src = edge_index[0].astype(jnp.int32)
    dst = edge_index[1].astype(jnp.int32)
    pad = EP - E
    srcp = jnp.concatenate([src, jnp.zeros((pad,), jnp.int32)])
    dstp = jnp.concatenate([dst, jnp.zeros((pad,), jnp.int32)])
    attrp = jnp.concatenate([edge_attr, jnp.zeros((pad,), jnp.float32)])

    degs = _deg_call(dstp, attrp)
    dis = _dis_call(degs)

    g1 = _mm1(x, W1, dis)
    p = _agg_call(g1, srcp, dstp, attrp)
    g2 = _mm2(p, dis, b1.reshape(1, D), W2)
    q = _agg_call(g2, srcp, dstp, attrp)
    out = _fin(q, dis, b2.reshape(1, D))
    return jnp.stack((out,))
